# baseline (speedup 1.0000x reference)
import functools
import math

import jax
import jax.numpy as jnp
from jax import lax
from jax.experimental import pallas as pl
from jax.experimental.pallas import tpu as pltpu
from jax.experimental.pallas import tpu_sc as plsc

_SCALE = 16
_RMAX = 3
_HL = 14
_WL = 14
_C = 64
_NT = _HL * _WL
_OFFS = tuple((dy, dx) for dy in range(-_RMAX, _RMAX + 1)
              for dx in range(-_RMAX, _RMAX + 1)
              if dy * dy + dx * dx <= _RMAX * _RMAX)
_K = len(_OFFS)
_NW = 32
_TPW = -(-_NT // _NW)
_NEG = -1e30


def _resize_mat(Hi, Ho):
    import numpy as np
    ys = np.maximum((np.arange(Ho, dtype=np.float64) + 0.5) * (Hi / Ho) - 0.5,
                    0.0)
    y0 = np.clip(np.floor(ys).astype(np.int64), 0, Hi - 1)
    y1 = np.minimum(y0 + 1, Hi - 1)
    wy = (ys - y0).astype(np.float32)
    W = np.zeros((Ho, Hi), np.float32)
    W[np.arange(Ho), y0] += 1.0 - wy
    W[np.arange(Ho), y1] += wy
    return jnp.asarray(W)


def _bilinear_resize(img, Ho, Wo):
    B, C, Hi, Wi = img.shape
    Wy = _resize_mat(Hi, Ho)
    Wx = _resize_mat(Wi, Wo)
    return jnp.einsum("oh,bchw,pw->bcop", Wy, img, Wx,
                      precision=jax.lax.Precision.HIGHEST)


def _splat_i32(v):
    return jnp.full((16,), v, jnp.int32)


def _jbu_sc_body(pv_h, ght_h, r2_h, s_h, invd_h,
                 pv, ght, r2v, wbuf, mbuf, dbuf):
    wid = lax.axis_index("s") * 2 + lax.axis_index("c")
    pltpu.sync_copy(pv_h, pv)
    col = [_splat_i32(c) for c in range(8)]
    xio = lax.iota(jnp.int32, 16).astype(jnp.float32)
    zv = jnp.zeros((16,), jnp.float32)
    for kp in range(_K, 32):
        for r16 in range(16):
            wbuf[kp, pl.ds(r16 * 16, 16)] = zv

    def do_tile(tile):
        i = tile // _WL
        j = tile - i * _WL
        pltpu.sync_copy(ght_h.at[tile], ght)
        pltpu.sync_copy(r2_h.at[tile], r2v)
        yb = (i * 16).astype(jnp.float32)
        xb = (j * 16).astype(jnp.float32)

        for k, (dy, dx) in enumerate(_OFFS):
            ni = jnp.clip(i + dy, 0, _HL - 1)
            nj = jnp.clip(j + dx, 0, _WL - 1)
            f = ni * _WL + nj
            fsp = _splat_i32(f)
            a = plsc.load_gather(pv, [fsp, col[0]])
            b = plsc.load_gather(pv, [fsp, col[1]])
            ct = plsc.load_gather(pv, [fsp, col[2]])
            st = plsc.load_gather(pv, [fsp, col[3]])
            cr = plsc.load_gather(pv, [fsp, col[4]])
            g0 = plsc.load_gather(pv, [fsp, col[5]])
            g1 = plsc.load_gather(pv, [fsp, col[6]])
            g2 = plsc.load_gather(pv, [fsp, col[7]])
            cxs = (nj.astype(jnp.float32) + 0.5) * float(_SCALE) - 0.5
            cys = (ni.astype(jnp.float32) + 0.5) * float(_SCALE) - 0.5
            dxv = xio + (xb - cxs)
            rad2 = float(dy * dy + dx * dx)

            def row1(r, _, k=k, dxv=dxv, a=a, b=b, ct=ct, st=st, cr=cr,
                     g0=g0, g1=g1, g2=g2, cys=cys):
                p = r * 16
                dyv = yb + r.astype(jnp.float32) - cys
                xp = dxv * ct + dyv * st
                yp = dyv * ct - dxv * st
                gh0 = ght[pl.ds(p, 16)]
                gh1 = ght[pl.ds(256 + p, 16)]
                gh2 = ght[pl.ds(512 + p, 16)]
                e0 = gh0 - g0
                e1 = gh1 - g1
                e2 = gh2 - g2
                d2 = e0 * e0 + e1 * e1 + e2 * e2
                lw = -(xp * xp * a + yp * yp * b) - d2 * cr
                mask = r2v[pl.ds(p, 16)] >= rad2
                lwm = jnp.where(mask, lw, _NEG)
                wbuf[k, pl.ds(p, 16)] = lwm
                if k == 0:
                    mbuf[pl.ds(p, 16)] = lwm
                else:
                    mbuf[pl.ds(p, 16)] = jnp.maximum(mbuf[pl.ds(p, 16)], lwm)
                return 0

            lax.fori_loop(0, 16, row1, 0, unroll=False)

        def row2(r, _):
            p = r * 16
            m = mbuf[pl.ds(p, 16)]
            den = jnp.zeros((16,), jnp.float32)
            for k in range(_K):
                s = jnp.exp(wbuf[k, pl.ds(p, 16)] - m)
                wbuf[k, pl.ds(p, 16)] = s
                den = den + s
            dbuf[pl.ds(p, 16)] = 1.0 / den
            return 0

        lax.fori_loop(0, 16, row2, 0, unroll=False)

        pltpu.sync_copy(wbuf, s_h.at[tile])
        pltpu.sync_copy(dbuf, invd_h.at[tile])

    def tloop(t, _):
        tile = wid + _NW * t

        @pl.when(tile < _NT)
        def _():
            do_tile(tile)

        return 0

    lax.fori_loop(0, _TPW, tloop, 0, unroll=False)


@jax.jit
def kernel(feat_lr, guide_hr, sx_raw, sy_raw, th_raw, sr_raw):
    B, C, Hl, Wl = feat_lr.shape
    _, _, Hh, Wh = guide_hr.shape
    sigma_x = jnp.exp(sx_raw)
    sigma_y = jnp.exp(sy_raw)
    theta = math.pi * jnp.tanh(th_raw)
    sigma_r = jnp.exp(sr_raw)
    sx = jnp.maximum(sigma_x, 1e-6)[0, 0]
    sy = jnp.maximum(sigma_y, 1e-6)[0, 0]
    sr = jnp.maximum(sigma_r, 1e-6)[0, 0]
    a_m = 1.0 / (2.0 * sx * sx + 1e-8)
    b_m = 1.0 / (2.0 * sy * sy + 1e-8)
    cr_m = 1.0 / (2.0 * sr * sr + 1e-8)
    cos_m = jnp.cos(theta[0, 0])
    sin_m = jnp.sin(theta[0, 0])
    glr = _bilinear_resize(guide_hr, Hl, Wl)[0]
    zer = jnp.zeros((Hl, Wl), jnp.float32)
    pv = jnp.stack([a_m, b_m, cos_m, sin_m, cr_m, glr[0], glr[1], glr[2]]
                   + [zer] * 8, axis=-1).reshape(_NT, 16)
    sigma_eff = jnp.maximum(sigma_x, sigma_y)[0, 0]
    Wy = _resize_mat(Hl, Hh).reshape(Hl, 16, Hl)
    Wx = _resize_mat(Wl, Wh).reshape(Wl, 16, Wl)
    sig_t = jnp.einsum("iyh,hw,jxw->ijyx", Wy, sigma_eff, Wx,
                       precision=jax.lax.Precision.HIGHEST)
    R_map = jnp.clip(jnp.ceil(2.0 * sig_t), 1, _RMAX)
    r2t = (R_map * R_map).astype(jnp.float32).reshape(_NT, 256)
    ght = (guide_hr[0].reshape(3, Hl, 16, Wl, 16)
           .transpose(1, 3, 0, 2, 4).reshape(_NT, 3 * 256))
    fpad = jnp.pad(feat_lr[0], ((0, 0), (_RMAX, _RMAX), (_RMAX, _RMAX)),
                   mode="edge")
    fg = jnp.stack([fpad[:, _RMAX + dy:_RMAX + dy + Hl,
                         _RMAX + dx:_RMAX + dx + Wl]
                    for dy, dx in _OFFS]
                   + [jnp.zeros((C, Hl, Wl), jnp.float32)] * (32 - _K),
                   axis=0)
    featgT = fg.transpose(2, 3, 1, 0).reshape(_NT, _C, 32)

    mesh = plsc.VectorSubcoreMesh(core_axis_name="c", subcore_axis_name="s",
                                  num_cores=2, num_subcores=16)
    s_t, invd = pl.kernel(
        _jbu_sc_body,
        mesh=mesh,
        compiler_params=pltpu.CompilerParams(needs_layout_passes=False,
                                             use_tc_tiling_on_sc=True),
        out_type=[
            jax.ShapeDtypeStruct((_NT, 32, 256), jnp.float32),
            jax.ShapeDtypeStruct((_NT, 256), jnp.float32),
        ],
        scratch_types=[
            pltpu.VMEM((_NT, 16), jnp.float32),
            pltpu.VMEM((3 * 256,), jnp.float32),
            pltpu.VMEM((256,), jnp.float32),
            pltpu.VMEM((32, 256), jnp.float32),
            pltpu.VMEM((256,), jnp.float32),
            pltpu.VMEM((256,), jnp.float32),
        ],
    )(pv, ght, r2t)

    def _mm_body(fg_ref, s_ref, iv_ref, o_ref):
        num = jax.lax.dot_general(
            fg_ref[...], s_ref[...], (((2,), (1,)), ((0,), (0,))),
            precision=jax.lax.Precision.HIGHEST)
        o_ref[...] = num * iv_ref[...]

    TB = 14
    out_t = pl.pallas_call(
        _mm_body,
        grid=(_NT // TB,),
        in_specs=[
            pl.BlockSpec((TB, _C, 32), lambda g: (g, 0, 0)),
            pl.BlockSpec((TB, 32, 256), lambda g: (g, 0, 0)),
            pl.BlockSpec((TB, 1, 256), lambda g: (g, 0, 0)),
        ],
        out_specs=pl.BlockSpec((TB, _C, 256), lambda g: (g, 0, 0)),
        out_shape=jax.ShapeDtypeStruct((_NT, _C, 256), jnp.float32),
    )(featgT, s_t, invd[:, None, :])
    out = (out_t.reshape(Hl, Wl, _C, 16, 16)
           .transpose(2, 0, 3, 1, 4).reshape(1, _C, Hh, Wh))
    return out


# speedup vs baseline: 1.8370x; 1.8370x over previous
import functools
import math

import jax
import jax.numpy as jnp
from jax import lax
from jax.experimental import pallas as pl
from jax.experimental.pallas import tpu as pltpu
from jax.experimental.pallas import tpu_sc as plsc

_SCALE = 16
_RMAX = 3
_HL = 14
_WL = 14
_C = 64
_NT = _HL * _WL
_OFFS = tuple((dy, dx) for dy in range(-_RMAX, _RMAX + 1)
              for dx in range(-_RMAX, _RMAX + 1)
              if dy * dy + dx * dx <= _RMAX * _RMAX)
_K = len(_OFFS)
_NW = 32
_TPW = -(-_NT // _NW)
_NEG = -1e30


def _resize_mat(Hi, Ho):
    import numpy as np
    ys = np.maximum((np.arange(Ho, dtype=np.float64) + 0.5) * (Hi / Ho) - 0.5,
                    0.0)
    y0 = np.clip(np.floor(ys).astype(np.int64), 0, Hi - 1)
    y1 = np.minimum(y0 + 1, Hi - 1)
    wy = (ys - y0).astype(np.float32)
    W = np.zeros((Ho, Hi), np.float32)
    W[np.arange(Ho), y0] += 1.0 - wy
    W[np.arange(Ho), y1] += wy
    return jnp.asarray(W)


def _bilinear_resize(img, Ho, Wo):
    B, C, Hi, Wi = img.shape
    Wy = _resize_mat(Hi, Ho)
    Wx = _resize_mat(Wi, Wo)
    return jnp.einsum("oh,bchw,pw->bcop", Wy, img, Wx,
                      precision=jax.lax.Precision.HIGHEST)


def _splat_i32(v):
    return jnp.full((16,), v, jnp.int32)


def _gather_mat():
    import numpy as np
    G = np.zeros((_NT * 32, _NT), np.float32)
    for i in range(_HL):
        for j in range(_WL):
            t = i * _WL + j
            for k, (dy, dx) in enumerate(_OFFS):
                ni = min(max(i + dy, 0), _HL - 1)
                nj = min(max(j + dx, 0), _WL - 1)
                G[t * 32 + k, ni * _WL + nj] = 1.0
    return jnp.asarray(G)


def _jbu_sc_body(pv_h, ght_h, r2_h, s_h, invd_h,
                 pv, ght, r2v, wbuf, mbuf, dbuf):
    wid = lax.axis_index("s") * 2 + lax.axis_index("c")
    pltpu.sync_copy(pv_h, pv)
    col = [_splat_i32(c) for c in range(8)]
    xio = lax.iota(jnp.int32, 16).astype(jnp.float32)
    zv = jnp.zeros((16,), jnp.float32)
    for kp in range(_K, 32):
        for r16 in range(16):
            wbuf[kp, pl.ds(r16 * 16, 16)] = zv

    def do_tile(tile):
        i = tile // _WL
        j = tile - i * _WL
        pltpu.sync_copy(ght_h.at[tile], ght)
        pltpu.sync_copy(r2_h.at[tile], r2v)
        yb = (i * 16).astype(jnp.float32)
        xb = (j * 16).astype(jnp.float32)

        for k, (dy, dx) in enumerate(_OFFS):
            ni = jnp.clip(i + dy, 0, _HL - 1)
            nj = jnp.clip(j + dx, 0, _WL - 1)
            f = ni * _WL + nj
            fsp = _splat_i32(f)
            a = plsc.load_gather(pv, [fsp, col[0]])
            b = plsc.load_gather(pv, [fsp, col[1]])
            ct = plsc.load_gather(pv, [fsp, col[2]])
            st = plsc.load_gather(pv, [fsp, col[3]])
            cr = plsc.load_gather(pv, [fsp, col[4]])
            g0 = plsc.load_gather(pv, [fsp, col[5]])
            g1 = plsc.load_gather(pv, [fsp, col[6]])
            g2 = plsc.load_gather(pv, [fsp, col[7]])
            cxs = (nj.astype(jnp.float32) + 0.5) * float(_SCALE) - 0.5
            cys = (ni.astype(jnp.float32) + 0.5) * float(_SCALE) - 0.5
            dxv = xio + (xb - cxs)
            rad2 = float(dy * dy + dx * dx)

            def row1(r, _, k=k, dxv=dxv, a=a, b=b, ct=ct, st=st, cr=cr,
                     g0=g0, g1=g1, g2=g2, cys=cys):
                p = r * 16
                dyv = yb + r.astype(jnp.float32) - cys
                xp = dxv * ct + dyv * st
                yp = dyv * ct - dxv * st
                gh0 = ght[pl.ds(p, 16)]
                gh1 = ght[pl.ds(256 + p, 16)]
                gh2 = ght[pl.ds(512 + p, 16)]
                e0 = gh0 - g0
                e1 = gh1 - g1
                e2 = gh2 - g2
                d2 = e0 * e0 + e1 * e1 + e2 * e2
                lw = -(xp * xp * a + yp * yp * b) - d2 * cr
                mask = r2v[pl.ds(p, 16)] >= rad2
                lwm = jnp.where(mask, lw, _NEG)
                wbuf[k, pl.ds(p, 16)] = lwm
                if k == 0:
                    mbuf[pl.ds(p, 16)] = lwm
                else:
                    mbuf[pl.ds(p, 16)] = jnp.maximum(mbuf[pl.ds(p, 16)], lwm)
                return 0

            lax.fori_loop(0, 16, row1, 0, unroll=False)

        def row2(r, _):
            p = r * 16
            m = mbuf[pl.ds(p, 16)]
            den = jnp.zeros((16,), jnp.float32)
            for k in range(_K):
                s = jnp.exp(wbuf[k, pl.ds(p, 16)] - m)
                wbuf[k, pl.ds(p, 16)] = s
                den = den + s
            dbuf[pl.ds(p, 16)] = 1.0 / den
            return 0

        lax.fori_loop(0, 16, row2, 0, unroll=False)

        pltpu.sync_copy(wbuf, s_h.at[tile])
        pltpu.sync_copy(dbuf, invd_h.at[tile])

    def tloop(t, _):
        tile = wid + _NW * t

        @pl.when(tile < _NT)
        def _():
            do_tile(tile)

        return 0

    lax.fori_loop(0, _TPW, tloop, 0, unroll=False)


@jax.jit
def kernel(feat_lr, guide_hr, sx_raw, sy_raw, th_raw, sr_raw):
    B, C, Hl, Wl = feat_lr.shape
    _, _, Hh, Wh = guide_hr.shape
    sigma_x = jnp.exp(sx_raw)
    sigma_y = jnp.exp(sy_raw)
    theta = math.pi * jnp.tanh(th_raw)
    sigma_r = jnp.exp(sr_raw)
    sx = jnp.maximum(sigma_x, 1e-6)[0, 0]
    sy = jnp.maximum(sigma_y, 1e-6)[0, 0]
    sr = jnp.maximum(sigma_r, 1e-6)[0, 0]
    a_m = 1.0 / (2.0 * sx * sx + 1e-8)
    b_m = 1.0 / (2.0 * sy * sy + 1e-8)
    cr_m = 1.0 / (2.0 * sr * sr + 1e-8)
    cos_m = jnp.cos(theta[0, 0])
    sin_m = jnp.sin(theta[0, 0])
    glr = _bilinear_resize(guide_hr, Hl, Wl)[0]
    zer = jnp.zeros((Hl, Wl), jnp.float32)
    pv = jnp.stack([a_m, b_m, cos_m, sin_m, cr_m, glr[0], glr[1], glr[2]]
                   + [zer] * 8, axis=-1).reshape(_NT, 16)
    sigma_eff = jnp.maximum(sigma_x, sigma_y)[0, 0]
    Wy = _resize_mat(Hl, Hh).reshape(Hl, 16, Hl)
    Wx = _resize_mat(Wl, Wh).reshape(Wl, 16, Wl)
    sig_t = jnp.einsum("iyh,hw,jxw->ijyx", Wy, sigma_eff, Wx,
                       precision=jax.lax.Precision.HIGHEST)
    R_map = jnp.clip(jnp.ceil(2.0 * sig_t), 1, _RMAX)
    r2t = (R_map * R_map).astype(jnp.float32).reshape(_NT, 256)
    ght = (guide_hr[0].reshape(3, Hl, 16, Wl, 16)
           .transpose(1, 3, 0, 2, 4).reshape(_NT, 3 * 256))
    featT = feat_lr[0].reshape(_C, _NT).T
    featg = jnp.dot(_gather_mat(), featT,
                    precision=jax.lax.Precision.HIGHEST)
    featg = featg.reshape(_NT, 32, _C)

    mesh = plsc.VectorSubcoreMesh(core_axis_name="c", subcore_axis_name="s",
                                  num_cores=2, num_subcores=16)
    s_t, invd = pl.kernel(
        _jbu_sc_body,
        mesh=mesh,
        compiler_params=pltpu.CompilerParams(needs_layout_passes=False,
                                             use_tc_tiling_on_sc=True),
        out_type=[
            jax.ShapeDtypeStruct((_NT, 32, 256), jnp.float32),
            jax.ShapeDtypeStruct((_NT, 256), jnp.float32),
        ],
        scratch_types=[
            pltpu.VMEM((_NT, 16), jnp.float32),
            pltpu.VMEM((3 * 256,), jnp.float32),
            pltpu.VMEM((256,), jnp.float32),
            pltpu.VMEM((32, 256), jnp.float32),
            pltpu.VMEM((256,), jnp.float32),
            pltpu.VMEM((256,), jnp.float32),
        ],
    )(pv, ght, r2t)

    def _mm_body(fg_ref, s_ref, iv_ref, o_ref):
        num = jax.lax.dot_general(
            fg_ref[...], s_ref[...], (((1,), (1,)), ((0,), (0,))),
            precision=jax.lax.Precision.HIGHEST)
        o_ref[...] = num * iv_ref[...]

    TB = 14
    out_t = pl.pallas_call(
        _mm_body,
        grid=(_NT // TB,),
        in_specs=[
            pl.BlockSpec((TB, 32, _C), lambda g: (g, 0, 0)),
            pl.BlockSpec((TB, 32, 256), lambda g: (g, 0, 0)),
            pl.BlockSpec((TB, 1, 256), lambda g: (g, 0, 0)),
        ],
        out_specs=pl.BlockSpec((TB, _C, 256), lambda g: (g, 0, 0)),
        out_shape=jax.ShapeDtypeStruct((_NT, _C, 256), jnp.float32),
    )(featg, s_t, invd[:, None, :])
    out = (out_t.reshape(Hl, Wl, _C, 16, 16)
           .transpose(2, 0, 3, 1, 4).reshape(1, _C, Hh, Wh))
    return out


# speedup vs baseline: 1.9228x; 1.0467x over previous
import functools
import math

import jax
import jax.numpy as jnp
from jax import lax
from jax.experimental import pallas as pl
from jax.experimental.pallas import tpu as pltpu
from jax.experimental.pallas import tpu_sc as plsc

_SCALE = 16
_RMAX = 3
_HL = 14
_WL = 14
_C = 64
_NT = _HL * _WL
_OFFS = tuple((dy, dx) for dy in range(-_RMAX, _RMAX + 1)
              for dx in range(-_RMAX, _RMAX + 1)
              if dy * dy + dx * dx <= _RMAX * _RMAX)
_K = len(_OFFS)
_NW = 32
_TPW = -(-_NT // _NW)
_NEG = -1e30


def _resize_mat(Hi, Ho):
    import numpy as np
    ys = np.maximum((np.arange(Ho, dtype=np.float64) + 0.5) * (Hi / Ho) - 0.5,
                    0.0)
    y0 = np.clip(np.floor(ys).astype(np.int64), 0, Hi - 1)
    y1 = np.minimum(y0 + 1, Hi - 1)
    wy = (ys - y0).astype(np.float32)
    W = np.zeros((Ho, Hi), np.float32)
    W[np.arange(Ho), y0] += 1.0 - wy
    W[np.arange(Ho), y1] += wy
    return jnp.asarray(W)


def _bilinear_resize(img, Ho, Wo):
    B, C, Hi, Wi = img.shape
    Wy = _resize_mat(Hi, Ho)
    Wx = _resize_mat(Wi, Wo)
    return jnp.einsum("oh,bchw,pw->bcop", Wy, img, Wx,
                      precision=jax.lax.Precision.HIGHEST)


def _splat_i32(v):
    return jnp.full((16,), v, jnp.int32)


def _gather_mat():
    import numpy as np
    G = np.zeros((_NT * 32, _NT), np.float32)
    for i in range(_HL):
        for j in range(_WL):
            t = i * _WL + j
            for k, (dy, dx) in enumerate(_OFFS):
                ni = min(max(i + dy, 0), _HL - 1)
                nj = min(max(j + dx, 0), _WL - 1)
                G[t * 32 + k, ni * _WL + nj] = 1.0
    return jnp.asarray(G)


def _jbu_sc_body(pv_h, ght_h, r2_h, s_h, invd_h,
                 pv, ght2, r2v2, wbuf2, mbuf, dbuf2, insem, outsem):
    wid = lax.axis_index("s") * 2 + lax.axis_index("c")
    pltpu.sync_copy(pv_h, pv)
    col = [_splat_i32(c) for c in range(8)]
    xio = lax.iota(jnp.int32, 16).astype(jnp.float32)
    zv = jnp.zeros((16,), jnp.float32)
    for sl in range(2):
        for kp in range(_K, 32):
            for r16 in range(16):
                wbuf2[sl, kp, pl.ds(r16 * 16, 16)] = zv

    def do_tile(tile, slot):
        i = tile // _WL
        j = tile - i * _WL
        yb = (i * 16).astype(jnp.float32)
        xb = (j * 16).astype(jnp.float32)

        for k, (dy, dx) in enumerate(_OFFS):
            ni = jnp.clip(i + dy, 0, _HL - 1)
            nj = jnp.clip(j + dx, 0, _WL - 1)
            f = ni * _WL + nj
            fsp = _splat_i32(f)
            a = plsc.load_gather(pv, [fsp, col[0]])
            b = plsc.load_gather(pv, [fsp, col[1]])
            ct = plsc.load_gather(pv, [fsp, col[2]])
            st = plsc.load_gather(pv, [fsp, col[3]])
            cr = plsc.load_gather(pv, [fsp, col[4]])
            g0 = plsc.load_gather(pv, [fsp, col[5]])
            g1 = plsc.load_gather(pv, [fsp, col[6]])
            g2 = plsc.load_gather(pv, [fsp, col[7]])
            cxs = (nj.astype(jnp.float32) + 0.5) * float(_SCALE) - 0.5
            cys = (ni.astype(jnp.float32) + 0.5) * float(_SCALE) - 0.5
            dxv = xio + (xb - cxs)
            rad2 = float(dy * dy + dx * dx)

            def row1(r, _, k=k, dxv=dxv, a=a, b=b, ct=ct, st=st, cr=cr,
                     g0=g0, g1=g1, g2=g2, cys=cys):
                p = r * 16
                dyv = yb + r.astype(jnp.float32) - cys
                xp = dxv * ct + dyv * st
                yp = dyv * ct - dxv * st
                gh0 = ght2[slot, pl.ds(p, 16)]
                gh1 = ght2[slot, pl.ds(256 + p, 16)]
                gh2 = ght2[slot, pl.ds(512 + p, 16)]
                e0 = gh0 - g0
                e1 = gh1 - g1
                e2 = gh2 - g2
                d2 = e0 * e0 + e1 * e1 + e2 * e2
                lw = -(xp * xp * a + yp * yp * b) - d2 * cr
                mask = r2v2[slot, pl.ds(p, 16)] >= rad2
                lwm = jnp.where(mask, lw, _NEG)
                wbuf2[slot, k, pl.ds(p, 16)] = lwm
                if k == 0:
                    mbuf[pl.ds(p, 16)] = lwm
                else:
                    mbuf[pl.ds(p, 16)] = jnp.maximum(mbuf[pl.ds(p, 16)], lwm)
                return 0

            lax.fori_loop(0, 16, row1, 0, unroll=False)

        def row2(r, _):
            p = r * 16
            m = mbuf[pl.ds(p, 16)]
            den = jnp.zeros((16,), jnp.float32)
            for k in range(_K):
                s = jnp.exp(wbuf2[slot, k, pl.ds(p, 16)] - m)
                wbuf2[slot, k, pl.ds(p, 16)] = s
                den = den + s
            dbuf2[slot, pl.ds(p, 16)] = 1.0 / den
            return 0

        lax.fori_loop(0, 16, row2, 0, unroll=False)

    def tloop(t, _):
        tile = wid + _NW * t
        slot = lax.rem(t, 2)

        @pl.when(tile < _NT)
        def _():
            pltpu.make_async_copy(ght_h.at[tile], ght2.at[slot],
                                  insem.at[slot]).wait()
            pltpu.make_async_copy(r2_h.at[tile], r2v2.at[slot],
                                  insem.at[slot]).wait()
            nxt = tile + _NW

            @pl.when(nxt < _NT)
            def _():
                pltpu.async_copy(ght_h.at[nxt], ght2.at[1 - slot],
                                 insem.at[1 - slot])
                pltpu.async_copy(r2_h.at[nxt], r2v2.at[1 - slot],
                                 insem.at[1 - slot])

            @pl.when(t >= 2)
            def _():
                pltpu.make_async_copy(wbuf2.at[slot], s_h.at[tile],
                                      outsem.at[slot]).wait()
                pltpu.make_async_copy(dbuf2.at[slot], invd_h.at[tile],
                                      outsem.at[slot]).wait()

            do_tile(tile, slot)
            pltpu.async_copy(wbuf2.at[slot], s_h.at[tile], outsem.at[slot])
            pltpu.async_copy(dbuf2.at[slot], invd_h.at[tile],
                             outsem.at[slot])

        return 0

    pltpu.async_copy(ght_h.at[wid], ght2.at[0], insem.at[0])
    pltpu.async_copy(r2_h.at[wid], r2v2.at[0], insem.at[0])
    lax.fori_loop(0, _TPW, tloop, 0, unroll=False)
    nt = (_NT - 1 - wid) // _NW + 1
    for back in (2, 1):
        lastslot = lax.rem(nt - back, 2)
        pltpu.make_async_copy(wbuf2.at[lastslot], s_h.at[0],
                              outsem.at[lastslot]).wait()
        pltpu.make_async_copy(dbuf2.at[lastslot], invd_h.at[0],
                              outsem.at[lastslot]).wait()


@jax.jit
def kernel(feat_lr, guide_hr, sx_raw, sy_raw, th_raw, sr_raw):
    B, C, Hl, Wl = feat_lr.shape
    _, _, Hh, Wh = guide_hr.shape
    sigma_x = jnp.exp(sx_raw)
    sigma_y = jnp.exp(sy_raw)
    theta = math.pi * jnp.tanh(th_raw)
    sigma_r = jnp.exp(sr_raw)
    sx = jnp.maximum(sigma_x, 1e-6)[0, 0]
    sy = jnp.maximum(sigma_y, 1e-6)[0, 0]
    sr = jnp.maximum(sigma_r, 1e-6)[0, 0]
    a_m = 1.0 / (2.0 * sx * sx + 1e-8)
    b_m = 1.0 / (2.0 * sy * sy + 1e-8)
    cr_m = 1.0 / (2.0 * sr * sr + 1e-8)
    cos_m = jnp.cos(theta[0, 0])
    sin_m = jnp.sin(theta[0, 0])
    glr = _bilinear_resize(guide_hr, Hl, Wl)[0]
    zer = jnp.zeros((Hl, Wl), jnp.float32)
    pv = jnp.stack([a_m, b_m, cos_m, sin_m, cr_m, glr[0], glr[1], glr[2]]
                   + [zer] * 8, axis=-1).reshape(_NT, 16)
    sigma_eff = jnp.maximum(sigma_x, sigma_y)[0, 0]
    Wy = _resize_mat(Hl, Hh).reshape(Hl, 16, Hl)
    Wx = _resize_mat(Wl, Wh).reshape(Wl, 16, Wl)
    sig_t = jnp.einsum("iyh,hw,jxw->ijyx", Wy, sigma_eff, Wx,
                       precision=jax.lax.Precision.HIGHEST)
    R_map = jnp.clip(jnp.ceil(2.0 * sig_t), 1, _RMAX)
    r2t = (R_map * R_map).astype(jnp.float32).reshape(_NT, 256)
    ght = (guide_hr[0].reshape(3, Hl, 16, Wl, 16)
           .transpose(1, 3, 0, 2, 4).reshape(_NT, 3 * 256))
    featT = feat_lr[0].reshape(_C, _NT).T
    featg = jnp.dot(_gather_mat(), featT,
                    precision=jax.lax.Precision.HIGHEST)
    featg = featg.reshape(_NT, 32, _C)

    mesh = plsc.VectorSubcoreMesh(core_axis_name="c", subcore_axis_name="s",
                                  num_cores=2, num_subcores=16)
    s_t, invd = pl.kernel(
        _jbu_sc_body,
        mesh=mesh,
        compiler_params=pltpu.CompilerParams(needs_layout_passes=False),
        out_type=[
            jax.ShapeDtypeStruct((_NT, 32, 256), jnp.float32),
            jax.ShapeDtypeStruct((_NT, 256), jnp.float32),
        ],
        scratch_types=[
            pltpu.VMEM((_NT, 16), jnp.float32),
            pltpu.VMEM((2, 3 * 256), jnp.float32),
            pltpu.VMEM((2, 256), jnp.float32),
            pltpu.VMEM((2, 32, 256), jnp.float32),
            pltpu.VMEM((256,), jnp.float32),
            pltpu.VMEM((2, 256), jnp.float32),
            pltpu.SemaphoreType.DMA((2,)),
            pltpu.SemaphoreType.DMA((2,)),
        ],
    )(pv, ght, r2t)

    def _mm_body(fg_ref, s_ref, iv_ref, o_ref):
        num = jax.lax.dot_general(
            fg_ref[...], s_ref[...], (((1,), (1,)), ((0,), (0,))),
            precision=jax.lax.Precision.HIGHEST)
        o_ref[...] = num * iv_ref[...]

    TB = 14
    out_t = pl.pallas_call(
        _mm_body,
        grid=(_NT // TB,),
        in_specs=[
            pl.BlockSpec((TB, 32, _C), lambda g: (g, 0, 0)),
            pl.BlockSpec((TB, 32, 256), lambda g: (g, 0, 0)),
            pl.BlockSpec((TB, 1, 256), lambda g: (g, 0, 0)),
        ],
        out_specs=pl.BlockSpec((TB, _C, 256), lambda g: (g, 0, 0)),
        out_shape=jax.ShapeDtypeStruct((_NT, _C, 256), jnp.float32),
    )(featg, s_t, invd[:, None, :])
    out = (out_t.reshape(Hl, Wl, _C, 16, 16)
           .transpose(2, 0, 3, 1, 4).reshape(1, _C, Hh, Wh))
    return out
